# Initial kernel scaffold; baseline (speedup 1.0000x reference)
import jax
import jax.numpy as jnp
from jax.experimental import pallas as pl


def kernel(node_features, senders, receivers, edge_features, W_kernel, W_bias, We_kernel, We_bias):
    raise NotImplementedError("write your pallas kernel here")


# speedup vs baseline: 2.4391x; 2.4391x over previous
import functools

import jax
import jax.numpy as jnp
from jax import lax
from jax.experimental import pallas as pl
from jax.experimental.pallas import tpu as pltpu
from jax.experimental.pallas import tpu_sc as plsc


def _node_proj_body(nf_ref, w_ref, b_ref, y_ref):
    y_ref[0] = jnp.dot(nf_ref[...], w_ref[...],
                       preferred_element_type=jnp.float32) + b_ref[0]


def _edge_proj_body(ef_ref, we_ref, be_ref, ep_ref):
    ep_ref[...] = jnp.dot(ef_ref[...], we_ref[...],
                          preferred_element_type=jnp.float32) + be_ref[...]


def _make_sc_kernel(N, E, D):
    H = D // 2
    NS = 16
    C = 80
    EPT = E // NS
    NCHUNK = EPT // C
    RPT = 624
    WB = 48
    REM = N - NS * RPT

    mesh = plsc.VectorSubcoreMesh(core_axis_name="c", subcore_axis_name="s")

    @functools.partial(
        pl.kernel,
        out_type=(
            jax.ShapeDtypeStruct((E, D), jnp.float32),
            jax.ShapeDtypeStruct((N, D), jnp.float32),
        ),
        mesh=mesh,
        scratch_types=[
            pltpu.VMEM((8, C), jnp.int32),
            pltpu.VMEM((C,), jnp.int32),
            pltpu.VMEM((C,), jnp.int32),
            pltpu.VMEM((C,), jnp.int32),
            pltpu.VMEM((C, H), jnp.float32),
            pltpu.VMEM((C, H), jnp.float32),
            pltpu.VMEM((C, H), jnp.float32),
            pltpu.VMEM((C, H), jnp.float32),
            pltpu.VMEM((WB, H), jnp.float32),
            pltpu.VMEM_SHARED((N, H), jnp.float32),
            pltpu.SemaphoreType.DMA,
        ],
    )
    def sck(y_hbm, ep_hbm, sr_hbm, edges_out, nodes_out,
            srbuf, qc, kc, vc, qbuf, kbuf, vbuf, epbuf, wb, acc, sem):
        c = lax.axis_index("c")
        s = lax.axis_index("s")
        cm = pl.multiple_of(c * H, H)

        qoff = (2 + c) * N
        koff = (4 + c) * N
        voff = (6 + c) * N

        nrow0 = s * RPT
        for i in range(RPT // WB):
            src = pl.multiple_of(c * N + nrow0 + i * WB, 8)
            dst = pl.multiple_of(nrow0 + i * WB, 8)
            pltpu.sync_copy(y_hbm.at[pl.ds(src, WB)], wb)
            pltpu.sync_copy(wb, acc.at[pl.ds(dst, WB)])

        @pl.when(s == NS - 1)
        def _init_rem():
            src = pl.multiple_of(c * N + NS * RPT, 8)
            pltpu.sync_copy(y_hbm.at[pl.ds(src, REM)], wb.at[pl.ds(0, REM)])
            pltpu.sync_copy(wb.at[pl.ds(0, REM)],
                            acc.at[pl.ds(NS * RPT, REM)])

        plsc.subcore_barrier()

        ebase = s * EPT

        def chunk_body(j, carry):
            e0 = pl.multiple_of(ebase + j * C, 8)
            pltpu.sync_copy(sr_hbm.at[s, j], srbuf)
            for t in range(C // 16):
                sl = pl.ds(t * 16, 16)
                sv = srbuf[0, sl]
                r = srbuf[1, sl]
                qc[sl] = r + qoff
                kc[sl] = sv + koff
                vc[sl] = sv + voff
            cp1 = pltpu.async_copy(y_hbm.at[qc], qbuf, sem)
            cp2 = pltpu.async_copy(y_hbm.at[kc], kbuf, sem)
            cp3 = pltpu.async_copy(y_hbm.at[vc], vbuf, sem)
            cp4 = pltpu.async_copy(
                ep_hbm.at[pl.ds(e0, C), pl.ds(cm, H)], epbuf, sem)
            cp1.wait()
            cp2.wait()
            cp3.wait()
            cp4.wait()

            def row_body(r, rcarry):
                for t in range(H // 16):
                    sl = pl.ds(t * 16, 16)
                    e = qbuf[r, sl] + kbuf[r, sl] + epbuf[r, sl]
                    qbuf[r, sl] = e
                    sig = 1.0 / (1.0 + jnp.exp(-e))
                    vbuf[r, sl] = sig * vbuf[r, sl]
                return rcarry

            lax.fori_loop(0, C, row_body, 0)
            pltpu.sync_copy(qbuf, edges_out.at[pl.ds(e0, C), pl.ds(cm, H)])
            pltpu.sync_copy(vbuf, acc.at[srbuf.at[1]], add=True)
            return carry

        lax.fori_loop(0, NCHUNK, chunk_body, 0)
        plsc.subcore_barrier()

        for i in range(RPT // WB):
            off = pl.multiple_of(nrow0 + i * WB, 8)
            pltpu.sync_copy(acc.at[pl.ds(off, WB)], wb)
            pltpu.sync_copy(wb, nodes_out.at[pl.ds(off, WB), pl.ds(cm, H)])

        @pl.when(s == NS - 1)
        def _wb_rem():
            off = pl.multiple_of(NS * RPT, 8)
            pltpu.sync_copy(acc.at[pl.ds(off, REM)], wb.at[pl.ds(0, REM)])
            pltpu.sync_copy(wb.at[pl.ds(0, REM)],
                            nodes_out.at[pl.ds(off, REM), pl.ds(cm, H)])

    return sck


def kernel(node_features, senders, receivers, edge_features,
           W_kernel, W_bias, We_kernel, We_bias):
    N, D = node_features.shape
    E = senders.shape[0]
    DE = edge_features.shape[1]
    H = D // 2

    BN = 1000
    y = pl.pallas_call(
        _node_proj_body,
        grid=(N // BN, 8),
        in_specs=[
            pl.BlockSpec((BN, D), lambda i, j: (i, 0)),
            pl.BlockSpec((D, H), lambda i, j: (0, j)),
            pl.BlockSpec((1, 1, H), lambda i, j: (j, 0, 0)),
        ],
        out_specs=pl.BlockSpec((1, BN, H), lambda i, j: (j, i, 0)),
        out_shape=jax.ShapeDtypeStruct((8, N, H), jnp.float32),
    )(node_features, W_kernel, W_bias.reshape(8, 1, H))
    y_flat = y.reshape(8 * N, H)

    BE = 2000
    ep = pl.pallas_call(
        _edge_proj_body,
        grid=(E // BE,),
        in_specs=[
            pl.BlockSpec((BE, DE), lambda i: (i, 0)),
            pl.BlockSpec((DE, D), lambda i: (0, 0)),
            pl.BlockSpec((1, D), lambda i: (0, 0)),
        ],
        out_specs=pl.BlockSpec((BE, D), lambda i: (i, 0)),
        out_shape=jax.ShapeDtypeStruct((E, D), jnp.float32),
    )(edge_features, We_kernel, We_bias.reshape(1, D))

    NS = 16
    C = 80
    NCHUNK = E // (NS * C)
    s4 = senders.astype(jnp.int32).reshape(NS, NCHUNK, 1, C)
    r4 = receivers.astype(jnp.int32).reshape(NS, NCHUNK, 1, C)
    sr = jnp.concatenate(
        [s4, r4, jnp.zeros((NS, NCHUNK, 6, C), jnp.int32)], axis=2)
    sck = _make_sc_kernel(N, E, D)
    edges, nodes = sck(y_flat, ep, sr)
    return (nodes, edges)
